# Initial kernel scaffold; baseline (speedup 1.0000x reference)
import jax
import jax.numpy as jnp
from jax.experimental import pallas as pl


def kernel(x):
    raise NotImplementedError("write your pallas kernel here")


# speedup vs baseline: 3.0226x; 3.0226x over previous
import functools

import numpy as np
import jax
import jax.numpy as jnp
from jax import lax
from jax.experimental import pallas as pl
from jax.experimental.pallas import tpu as pltpu
from jax.experimental.pallas import tpu_sc as plsc

_C, _H, _W, _B, _DIM = 3, 64, 64, 32, 64
_SEQ = _C * _H * _W


def _source_rows() -> np.ndarray:
    diagonals = [[] for _ in range(_H + _W - 1)]
    for i in range(_H):
        for j in range(_W):
            s = i + j
            if s % 2 == 0:
                diagonals[s].insert(0, (i, j))
            else:
                diagonals[s].append((i, j))
    triples = []
    for diag in diagonals:
        for ij in diag:
            for c in range(_C):
                triples.append((c,) + ij)
    a = np.array(triples, dtype=np.int64)
    d_of_k = (a[:, 0] * _H + a[:, 1]) * _W + a[:, 2]
    src = np.empty(_SEQ, dtype=np.int64)
    src[d_of_k] = np.arange(_SEQ)
    rows = src[None, :] * _B + np.arange(_B)[:, None]
    return rows.astype(np.int32).reshape(-1)


_ROWS = _source_rows()

_NW = 32
_CHUNK = 128
_TOTAL_CHUNKS = (_B * _SEQ) // _CHUNK
_CHUNKS_PER_W = _TOTAL_CHUNKS // _NW


@functools.partial(
    pl.kernel,
    out_type=jax.ShapeDtypeStruct((_B * _SEQ, _DIM), jnp.float32),
    mesh=plsc.VectorSubcoreMesh(core_axis_name="c", subcore_axis_name="s"),
    scratch_types=[
        pltpu.VMEM((_CHUNK,), jnp.int32),
        pltpu.VMEM((_CHUNK, _DIM), jnp.float32),
        pltpu.SemaphoreType.DMA,
    ],
    compiler_params=pltpu.CompilerParams(use_tc_tiling_on_sc=False),
)
def _zigzag_gather(x_hbm, idx_hbm, out_hbm, idx_v, rows_v, sem):
    w = lax.axis_index("s") * 2 + lax.axis_index("c")
    first = w * _CHUNKS_PER_W

    def body(ci, carry):
        base = (first + ci) * _CHUNK
        pltpu.sync_copy(idx_hbm.at[pl.ds(base, _CHUNK)], idx_v)
        pltpu.async_copy(x_hbm.at[idx_v], rows_v, sem).wait()
        pltpu.sync_copy(rows_v, out_hbm.at[pl.ds(base, _CHUNK)])
        return carry

    lax.fori_loop(0, _CHUNKS_PER_W, body, 0)


def kernel(x):
    xrows = x.reshape(_SEQ * _B, _DIM)
    idx = jnp.asarray(_ROWS)
    out = _zigzag_gather(xrows, idx)
    return out.reshape(_B, _C, _H, _W, _DIM)


# speedup vs baseline: 3.7821x; 1.2513x over previous
import functools

import numpy as np
import jax
import jax.numpy as jnp
from jax import lax
from jax.experimental import pallas as pl
from jax.experimental.pallas import tpu as pltpu
from jax.experimental.pallas import tpu_sc as plsc

_C, _H, _W, _B, _DIM = 3, 64, 64, 32, 64
_SEQ = _C * _H * _W


def _source_rows() -> np.ndarray:
    diagonals = [[] for _ in range(_H + _W - 1)]
    for i in range(_H):
        for j in range(_W):
            s = i + j
            if s % 2 == 0:
                diagonals[s].insert(0, (i, j))
            else:
                diagonals[s].append((i, j))
    triples = []
    for diag in diagonals:
        for ij in diag:
            for c in range(_C):
                triples.append((c,) + ij)
    a = np.array(triples, dtype=np.int64)
    d_of_k = (a[:, 0] * _H + a[:, 1]) * _W + a[:, 2]
    src = np.empty(_SEQ, dtype=np.int64)
    src[d_of_k] = np.arange(_SEQ)
    rows = src[None, :] * _B + np.arange(_B)[:, None]
    return rows.astype(np.int32).reshape(-1)


_ROWS = _source_rows()

_NW = 32
_CHUNK = 128
_TOTAL_CHUNKS = (_B * _SEQ) // _CHUNK
_CHUNKS_PER_W = _TOTAL_CHUNKS // _NW
_NBUF = 8
_DEPTH = 4
_NGROUP = _CHUNKS_PER_W // _NBUF


@functools.partial(
    pl.kernel,
    out_type=jax.ShapeDtypeStruct((_B * _SEQ, _DIM), jnp.float32),
    mesh=plsc.VectorSubcoreMesh(core_axis_name="c", subcore_axis_name="s"),
    scratch_types=[
        pltpu.VMEM((_CHUNKS_PER_W * _CHUNK,), jnp.int32),
        pltpu.VMEM((_NBUF, _CHUNK, _DIM), jnp.float32),
        pltpu.SemaphoreType.DMA((_NBUF,)),
        pltpu.SemaphoreType.DMA((_NBUF,)),
    ],
    compiler_params=pltpu.CompilerParams(use_tc_tiling_on_sc=False),
)
def _zigzag_gather(x_hbm, idx_hbm, out_hbm, idx_v, bufs_v, gsem, ssem):
    w = lax.axis_index("s") * 2 + lax.axis_index("c")
    first = w * _CHUNKS_PER_W * _CHUNK

    pltpu.sync_copy(idx_hbm.at[pl.ds(first, _CHUNKS_PER_W * _CHUNK)], idx_v)

    def gather(ci, b):
        return pltpu.async_copy(
            x_hbm.at[idx_v.at[pl.ds(ci * _CHUNK, _CHUNK)]],
            bufs_v.at[b],
            gsem.at[b],
        )

    def store(ci, b):
        return pltpu.async_copy(
            bufs_v.at[b],
            out_hbm.at[pl.ds(first + ci * _CHUNK, _CHUNK)],
            ssem.at[b],
        )

    def wait_gather(b):
        pltpu.make_async_copy(
            x_hbm.at[pl.ds(0, _CHUNK)], bufs_v.at[b], gsem.at[b]
        ).wait()

    def wait_store(b):
        pltpu.make_async_copy(
            bufs_v.at[b], out_hbm.at[pl.ds(first, _CHUNK)], ssem.at[b]
        ).wait()

    for b in range(_DEPTH):
        gather(b, b)

    def body(g, carry):
        for b in range(_NBUF):
            ci = g * _NBUF + b
            wait_gather(b)
            store(ci, b)
            bn = (b + _DEPTH) % _NBUF
            if b < _DEPTH:
                @pl.when(g > 0)
                def _():
                    wait_store(bn)

                gather(ci + _DEPTH, bn)
            else:
                @pl.when(g < _NGROUP - 1)
                def _():
                    wait_store(bn)
                    gather(ci + _DEPTH, bn)
        return carry

    lax.fori_loop(0, _NGROUP, body, 0)

    for b in range(_NBUF):
        wait_store(b)


def kernel(x):
    xrows = x.reshape(_SEQ * _B, _DIM)
    idx = jnp.asarray(_ROWS)
    out = _zigzag_gather(xrows, idx)
    return out.reshape(_B, _C, _H, _W, _DIM)
